# baseline (device time: 46000 ns/iter reference)
import jax
import jax.numpy as jnp
from jax import lax
from jax.experimental import pallas as pl
from jax.experimental.pallas import tpu as pltpu

N_DEV = 4
B_SH = 64
B = N_DEV * B_SH
HALF = B // 2
D = 512
N_RDMA = 14


def kernel(x, Win0, Wout0, Win1, Wout1, Win2, Wout2):
    def body(x_ref, win0_ref, wout0_ref, win1_ref, wout1_ref, win2_ref,
             wout2_ref, out_ref, xfull_ref, part_ref, ag_ref, arh_ref,
             rs_ref, send_sems, recv_sems):
        my = lax.axis_index("i")
        y_p = my ^ 1
        x_p = 3 - my
        d_p = (3 - my) ^ 1

        barrier_sem = pltpu.get_barrier_semaphore()
        for nbr in (y_p, x_p):
            pl.semaphore_signal(
                barrier_sem, inc=1,
                device_id=(nbr,), device_id_type=pl.DeviceIdType.MESH,
            )
        pl.semaphore_wait(barrier_sem, 2)

        sem_counter = [0]

        def rdma(src, dst, target):
            i = sem_counter[0]
            sem_counter[0] += 1
            return pltpu.make_async_remote_copy(
                src_ref=src, dst_ref=dst,
                send_sem=send_sems.at[i], recv_sem=recv_sems.at[i],
                device_id=(target,), device_id_type=pl.DeviceIdType.MESH,
            )

        def rows(c):
            return pl.ds(c * B_SH, B_SH)

        def mlp_chunk(v, win_ref, wout_ref):
            h = jnp.maximum(
                jnp.dot(v, win_ref[:, :], preferred_element_type=jnp.float32),
                0.0,
            )
            return jnp.dot(h, wout_ref[:, :],
                           preferred_element_type=jnp.float32)

        r0 = rdma(x_ref, ag_ref.at[0], y_p)
        r1 = rdma(x_ref, ag_ref.at[1], x_p)
        r0.start()
        r1.start()
        part_ref[rows(my), :] = mlp_chunk(x_ref[:, :], win0_ref, wout0_ref)
        r0.wait()
        r2 = rdma(ag_ref.at[0], ag_ref.at[2], x_p)
        r2.start()
        part_ref[rows(y_p), :] = mlp_chunk(ag_ref[0], win0_ref, wout0_ref)
        r1.wait()
        part_ref[rows(x_p), :] = mlp_chunk(ag_ref[1], win0_ref, wout0_ref)
        r2.wait()
        part_ref[rows(d_p), :] = mlp_chunk(ag_ref[2], win0_ref, wout0_ref)

        for bidx, (win_ref, wout_ref) in enumerate(
                ((win1_ref, wout1_ref), (win2_ref, wout2_ref))):
            a0 = 4 * bidx
            ds_a = pl.ds(0, HALF)
            ds_b = pl.ds(HALF, HALF)
            ra = rdma(part_ref.at[ds_a], arh_ref.at[a0], y_p)
            rb = rdma(part_ref.at[ds_b], arh_ref.at[a0 + 1], x_p)
            ra.start()
            rb.start()
            ra.wait()
            xfull_ref[ds_a, :] = part_ref[ds_a, :] + arh_ref[a0]
            ra2 = rdma(xfull_ref.at[ds_a], arh_ref.at[a0 + 2], x_p)
            ra2.start()
            rb.wait()
            xfull_ref[ds_b, :] = part_ref[ds_b, :] + arh_ref[a0 + 1]
            rb2 = rdma(xfull_ref.at[ds_b], arh_ref.at[a0 + 3], y_p)
            rb2.start()
            ra2.wait()
            xfull_ref[ds_a, :] = xfull_ref[ds_a, :] + arh_ref[a0 + 2]
            part_ref[ds_a, :] = mlp_chunk(xfull_ref[ds_a, :],
                                          win_ref, wout_ref)
            rb2.wait()
            xfull_ref[ds_b, :] = xfull_ref[ds_b, :] + arh_ref[a0 + 3]
            part_ref[ds_b, :] = mlp_chunk(xfull_ref[ds_b, :],
                                          win_ref, wout_ref)

        rq0 = rdma(part_ref.at[rows(y_p)], rs_ref.at[0], y_p)
        rq1 = rdma(part_ref.at[rows(x_p)], rs_ref.at[1], x_p)
        rq2 = rdma(part_ref.at[rows(d_p)], rs_ref.at[2], d_p)
        rq0.start()
        rq1.start()
        rq2.start()
        rq0.wait()
        rq1.wait()
        rq2.wait()
        out_ref[:, :] = (part_ref[rows(my), :] + rs_ref[0]
                         + rs_ref[1] + rs_ref[2])

    return pl.pallas_call(
        body,
        out_shape=jax.ShapeDtypeStruct((B_SH, D), jnp.float32),
        in_specs=[pl.BlockSpec(memory_space=pltpu.VMEM)] * 7,
        out_specs=pl.BlockSpec(memory_space=pltpu.VMEM),
        scratch_shapes=[
            pltpu.VMEM((B, D), jnp.float32),
            pltpu.VMEM((B, D), jnp.float32),
            pltpu.VMEM((3, B_SH, D), jnp.float32),
            pltpu.VMEM((8, HALF, D), jnp.float32),
            pltpu.VMEM((3, B_SH, D), jnp.float32),
            pltpu.SemaphoreType.DMA((N_RDMA,)),
            pltpu.SemaphoreType.DMA((N_RDMA,)),
        ],
        compiler_params=pltpu.CompilerParams(collective_id=0),
    )(x, Win0, Wout0, Win1, Wout1, Win2, Wout2)


# device time: 36044 ns/iter; 1.2762x vs baseline; 1.2762x over previous
import jax
import jax.numpy as jnp
from jax import lax
from jax.experimental import pallas as pl
from jax.experimental.pallas import tpu as pltpu

N_DEV = 4
B_SH = 64
B = N_DEV * B_SH
HALF = B // 2
D = 512
N_RDMA = 14

F32 = jnp.float32
BF16 = jnp.bfloat16


def kernel(x, Win0, Wout0, Win1, Wout1, Win2, Wout2):
    def body(x_ref, win0_ref, wout0_ref, win1_ref, wout1_ref, win2_ref,
             wout2_ref, out_ref, xfull_ref, part_ref, xb_ref, agb_ref,
             sab_ref, arb_ref, rsb_s_ref, rsb_r_ref, send_sems, recv_sems):
        my = lax.axis_index("i")
        y_p = my ^ 1
        x_p = 3 - my
        d_p = (3 - my) ^ 1

        barrier_sem = pltpu.get_barrier_semaphore()
        for nbr in (y_p, x_p):
            pl.semaphore_signal(
                barrier_sem, inc=1,
                device_id=(nbr,), device_id_type=pl.DeviceIdType.MESH,
            )
        pl.semaphore_wait(barrier_sem, 2)

        sem_counter = [0]

        def rdma(src, dst, target):
            i = sem_counter[0]
            sem_counter[0] += 1
            return pltpu.make_async_remote_copy(
                src_ref=src, dst_ref=dst,
                send_sem=send_sems.at[i], recv_sem=recv_sems.at[i],
                device_id=(target,), device_id_type=pl.DeviceIdType.MESH,
            )

        def rows(c):
            return pl.ds(c * B_SH, B_SH)

        def mlp_chunk(v, win_ref, wout_ref):
            h = jnp.maximum(
                jnp.dot(v, win_ref[:, :], preferred_element_type=F32), 0.0)
            return jnp.dot(h, wout_ref[:, :], preferred_element_type=F32)

        ds_a = pl.ds(0, HALF)
        ds_b = pl.ds(HALF, HALF)

        xb_ref[:, :] = x_ref[:, :].astype(BF16)
        r0 = rdma(xb_ref, agb_ref.at[0], y_p)
        r1 = rdma(xb_ref, agb_ref.at[1], x_p)
        r2 = rdma(xb_ref, agb_ref.at[2], d_p)
        r0.start()
        r1.start()
        r2.start()
        part_ref[rows(my), :] = mlp_chunk(x_ref[:, :], win0_ref, wout0_ref)
        r0.wait()
        part_ref[rows(y_p), :] = mlp_chunk(
            agb_ref[0].astype(F32), win0_ref, wout0_ref)
        r1.wait()
        part_ref[rows(x_p), :] = mlp_chunk(
            agb_ref[1].astype(F32), win0_ref, wout0_ref)
        r2.wait()
        part_ref[rows(d_p), :] = mlp_chunk(
            agb_ref[2].astype(F32), win0_ref, wout0_ref)

        sab_ref[0] = part_ref[ds_a, :].astype(BF16)
        ra = rdma(sab_ref.at[0], arb_ref.at[0], y_p)
        ra.start()
        sab_ref[1] = part_ref[ds_b, :].astype(BF16)
        rb = rdma(sab_ref.at[1], arb_ref.at[1], x_p)
        rb.start()
        ra.wait()
        xfull_ref[ds_a, :] = part_ref[ds_a, :] + arb_ref[0].astype(F32)
        sab_ref[2] = xfull_ref[ds_a, :].astype(BF16)
        ra2 = rdma(sab_ref.at[2], arb_ref.at[2], x_p)
        ra2.start()
        rb.wait()
        xfull_ref[ds_b, :] = part_ref[ds_b, :] + arb_ref[1].astype(F32)
        sab_ref[3] = xfull_ref[ds_b, :].astype(BF16)
        rb2 = rdma(sab_ref.at[3], arb_ref.at[3], y_p)
        rb2.start()
        ra2.wait()
        xfull_ref[ds_a, :] = xfull_ref[ds_a, :] + arb_ref[2].astype(F32)
        part_ref[ds_a, :] = mlp_chunk(xfull_ref[ds_a, :], win1_ref, wout1_ref)
        sab_ref[4] = part_ref[ds_a, :].astype(BF16)
        sa = rdma(sab_ref.at[4], arb_ref.at[4], y_p)
        sa.start()
        rb2.wait()
        xfull_ref[ds_b, :] = xfull_ref[ds_b, :] + arb_ref[3].astype(F32)
        part_ref[ds_b, :] = mlp_chunk(xfull_ref[ds_b, :], win1_ref, wout1_ref)
        sab_ref[5] = part_ref[ds_b, :].astype(BF16)
        sb = rdma(sab_ref.at[5], arb_ref.at[5], x_p)
        sb.start()
        sa.wait()
        xfull_ref[ds_a, :] = part_ref[ds_a, :] + arb_ref[4].astype(F32)
        sab_ref[6] = xfull_ref[ds_a, :].astype(BF16)
        sa2 = rdma(sab_ref.at[6], arb_ref.at[6], x_p)
        sa2.start()
        sb.wait()
        xfull_ref[ds_b, :] = part_ref[ds_b, :] + arb_ref[5].astype(F32)
        sab_ref[7] = xfull_ref[ds_b, :].astype(BF16)
        sb2 = rdma(sab_ref.at[7], arb_ref.at[7], y_p)
        sb2.start()
        sa2.wait()
        xfull_ref[ds_a, :] = xfull_ref[ds_a, :] + arb_ref[6].astype(F32)
        part_ref[ds_a, :] = mlp_chunk(xfull_ref[ds_a, :], win2_ref, wout2_ref)
        sb2.wait()
        xfull_ref[ds_b, :] = xfull_ref[ds_b, :] + arb_ref[7].astype(F32)
        part_ref[ds_b, :] = mlp_chunk(xfull_ref[ds_b, :], win2_ref, wout2_ref)

        rsb_s_ref[0] = part_ref[rows(y_p), :].astype(BF16)
        rq0 = rdma(rsb_s_ref.at[0], rsb_r_ref.at[0], y_p)
        rq0.start()
        rsb_s_ref[1] = part_ref[rows(x_p), :].astype(BF16)
        rq1 = rdma(rsb_s_ref.at[1], rsb_r_ref.at[1], x_p)
        rq1.start()
        rsb_s_ref[2] = part_ref[rows(d_p), :].astype(BF16)
        rq2 = rdma(rsb_s_ref.at[2], rsb_r_ref.at[2], d_p)
        rq2.start()
        rq0.wait()
        rq1.wait()
        rq2.wait()
        out_ref[:, :] = (part_ref[rows(my), :]
                         + rsb_r_ref[0].astype(F32)
                         + rsb_r_ref[1].astype(F32)
                         + rsb_r_ref[2].astype(F32))

    return pl.pallas_call(
        body,
        out_shape=jax.ShapeDtypeStruct((B_SH, D), F32),
        in_specs=[pl.BlockSpec(memory_space=pltpu.VMEM)] * 7,
        out_specs=pl.BlockSpec(memory_space=pltpu.VMEM),
        scratch_shapes=[
            pltpu.VMEM((B, D), F32),
            pltpu.VMEM((B, D), F32),
            pltpu.VMEM((B_SH, D), BF16),
            pltpu.VMEM((3, B_SH, D), BF16),
            pltpu.VMEM((8, HALF, D), BF16),
            pltpu.VMEM((8, HALF, D), BF16),
            pltpu.VMEM((3, B_SH, D), BF16),
            pltpu.VMEM((3, B_SH, D), BF16),
            pltpu.SemaphoreType.DMA((N_RDMA,)),
            pltpu.SemaphoreType.DMA((N_RDMA,)),
        ],
        compiler_params=pltpu.CompilerParams(collective_id=0),
    )(x, Win0, Wout0, Win1, Wout1, Win2, Wout2)
